# baseline (speedup 1.0000x reference)
import dataclasses

import jax
import jax.numpy as jnp
from jax import lax
from jax.experimental import pallas as pl
from jax.experimental.pallas import tpu as pltpu
from jax.experimental.pallas import tpu_sc as plsc

T = 4096
UP_LEN = T * 256
M = UP_LEN - 1
NC = 2
NS = 16
L = 16
NW = NC * NS
CHUNK = UP_LEN // NW
NVEC = CHUNK // L
NBLK = T // L

_mesh = plsc.VectorSubcoreMesh(core_axis_name="c", subcore_axis_name="s")


KCH = 8
CVEC = NVEC // KCH


def _sc_kernel(f0_hbm, out_hbm, f0_v, xv, yv, dv, rank_v, out_v):
    wid = lax.axis_index("s") * NC + lax.axis_index("c")
    pltpu.sync_copy(f0_hbm, f0_v)
    iota = lax.iota(jnp.int32, L)
    zero_i = jnp.zeros((L,), jnp.int32)
    lane0 = iota == 0

    def p1(c, carry):
        base = c * jnp.int32(L)
        vals = f0_v[pl.ds(base, L)]
        voiced = vals > 0.0
        v_i = voiced.astype(jnp.int32)
        rank_inc = jnp.cumsum(v_i) + carry
        rank_v[pl.ds(base, L)] = rank_inc - 1
        idx = jnp.maximum(rank_inc - 1, 0)
        pos_f = (iota + base).astype(jnp.float32)
        plsc.store_scatter(xv, [idx], pos_f, mask=voiced)
        plsc.store_scatter(yv, [idx], vals, mask=voiced)
        return jnp.max(rank_inc)

    n_v = lax.fori_loop(jnp.int32(0), jnp.int32(NBLK), p1, jnp.int32(0))

    def gat(ref, i_scalar):
        return plsc.load_gather(ref, [zero_i + i_scalar])

    base0 = wid * jnp.int32(CHUNK)
    nseg_sc = jnp.maximum(n_v - 2, 0)

    def q_of(i_s):
        p_s = i_s * jnp.int32(4095)
        hi_s = lax.shift_right_logical(p_s, jnp.int32(20))
        s_s = hi_s + (p_s & jnp.int32(M))
        return jnp.where(s_s >= M, hi_s + 1, hi_s)

    def seg_at(q_s):
        return jnp.clip(jnp.max(plsc.load_gather(rank_v, [zero_i + q_s])),
                        0, nseg_sc)

    slo_s = seg_at(q_of(base0))
    shi_s = seg_at(q_of(base0 + jnp.int32(CHUNK - 1)))
    b_lo = lax.shift_right_logical(slo_s, jnp.int32(4))
    b_hi = lax.shift_right_logical(shi_s + 1, jnp.int32(4)) + 1

    @plsc.parallel_loop(b_lo, b_hi, jnp.int32(1), unroll=1)
    def p2(c):
        base = c * jnp.int32(L)
        k = iota + base
        km1 = jnp.maximum(k - 1, 0)
        kp1 = jnp.minimum(k + 1, T - 1)
        xk = xv[pl.ds(base, L)]
        yk = yv[pl.ds(base, L)]
        xkm1 = plsc.load_gather(xv, [km1])
        xkp1 = plsc.load_gather(xv, [kp1])
        ykm1 = plsc.load_gather(yv, [km1])
        ykp1 = plsc.load_gather(yv, [kp1])
        dxk = xkp1 - xk
        dxkm1 = xk - xkm1
        dk = (ykp1 - yk) / dxk
        dkm1 = (yk - ykm1) / dxkm1
        w1 = 2.0 * dxk + dxkm1
        w2 = dxk + 2.0 * dxkm1
        mono = (dkm1 * dk) > 0.0
        skm1 = jnp.where(mono, dkm1, 1.0)
        sk = jnp.where(mono, dk, 1.0)
        hmean = (w1 + w2) / (w1 / skm1 + w2 / sk)
        dv[pl.ds(base, L)] = jnp.where(mono, hmean, 0.0)

    x0e = gat(xv, 0)
    x1e = gat(xv, 1)
    x2e = gat(xv, 2)
    y0e = gat(yv, 0)
    y1e = gat(yv, 1)
    y2e = gat(yv, 2)
    h0 = x1e - x0e
    h1 = x2e - x1e
    delta0 = (y1e - y0e) / h0
    delta1 = (y2e - y1e) / h1
    d0 = ((2.0 * h0 + h1) * delta0 - h0 * delta1) / (h0 + h1)
    mask0 = d0 * delta0 <= 0.0
    d0 = jnp.where(mask0, 0.0, d0)
    mask0b = (delta0 * delta1 < 0.0) & (~mask0)
    d0 = jnp.where(mask0b & (jnp.abs(d0) > 3.0 * jnp.abs(delta0)), 3.0 * delta0, d0)
    plsc.store_scatter(dv, [zero_i], d0, mask=lane0)

    m_i = jnp.maximum(n_v - 2, 0)
    mm1_i = jnp.maximum(n_v - 3, 0)
    xa = gat(xv, mm1_i)
    xb = gat(xv, jnp.minimum(mm1_i + 1, T - 1))
    xc = gat(xv, m_i)
    xd = gat(xv, jnp.minimum(m_i + 1, T - 1))
    ya = gat(yv, mm1_i)
    yb = gat(yv, jnp.minimum(mm1_i + 1, T - 1))
    yc = gat(yv, m_i)
    yd = gat(yv, jnp.minimum(m_i + 1, T - 1))
    hm2 = xb - xa
    hm1 = xd - xc
    dlast = (yd - yc) / hm1
    dlast2 = (yb - ya) / hm2
    dn = ((2.0 * hm1 + hm2) * dlast - hm1 * dlast2) / (hm1 + hm2)
    maskn = dn * dlast <= 0.0
    dn = jnp.where(maskn, 0.0, dn)
    masknb = (dlast * dlast2 < 0.0) & (~maskn)
    dn = jnp.where(masknb & (jnp.abs(dn) > 3.0 * jnp.abs(dlast)), 3.0 * dlast, dn)
    plsc.store_scatter(dv, [zero_i + jnp.maximum(n_v - 1, 0)], dn, mask=lane0)

    @pl.when(n_v == 2)
    def _():
        plsc.store_scatter(dv, [jnp.minimum(iota, 1)], delta0, mask=iota < 2)

    nseg_s = zero_i + nseg_sc
    nv_ok_b = (zero_i + (n_v >= 2).astype(jnp.int32)) == 1
    m_f = jnp.float32(M)

    def run_span(lo_vec, hi_vec):
        @plsc.parallel_loop(lo_vec, hi_vec, jnp.int32(1), unroll=6)
        def body(c):
            i = iota + (base0 + c * jnp.int32(L))
            p = i * 4095
            hi = lax.shift_right_logical(p, jnp.int32(20))
            lo = p & M
            sfold = hi + lo
            ge = sfold >= M
            r = jnp.where(ge, sfold - M, sfold)
            q = jnp.where(ge, hi + 1, hi)
            frac = r.astype(jnp.float32) / m_f
            seg = jnp.clip(plsc.load_gather(rank_v, [q]), 0, nseg_s)
            segp = seg + 1
            x0 = plsc.load_gather(xv, [seg])
            x1 = plsc.load_gather(xv, [segp])
            y0 = plsc.load_gather(yv, [seg])
            y1 = plsc.load_gather(yv, [segp])
            d0g = plsc.load_gather(dv, [seg])
            d1g = plsc.load_gather(dv, [segp])
            qf = q.astype(jnp.float32)
            h = x1 - x0
            t = ((qf - x0) + frac) / h
            hd0 = h * d0g
            hd1 = h * d1g
            dy = y1 - y0
            cc = 3.0 * dy - 2.0 * hd0 - hd1
            ee = hd0 + hd1 - 2.0 * dy
            up = ((ee * t + cc) * t + hd0) * t + y0
            up = jnp.where(up < 0.0, 0.0, up)
            ux = qf + frac
            fi = ux.astype(jnp.int32)
            fr = ux - fi.astype(jnp.float32)
            gt = fr > 0.5
            tie_odd = (fr == 0.5) & ((fi & 1) == 1)
            back = (fi + (gt | tie_odd).astype(jnp.int32)).astype(jnp.float32)
            keep = ((back == x0) | (back == x1)) & nv_ok_b
            out_v[pl.ds(c * jnp.int32(L), L)] = jnp.where(keep, up, 0.0)

    run_span(jnp.int32(0), jnp.int32(NVEC))
    pltpu.sync_copy(out_v, out_hbm.at[pl.ds(base0, CHUNK)])


def _compiler_params():
    cp = pltpu.CompilerParams()
    if "needs_layout_passes" in pltpu.CompilerParams.__dataclass_fields__:
        cp = dataclasses.replace(cp, needs_layout_passes=False)
    return cp


def _make_call():
    return pl.kernel(
        _sc_kernel,
        out_type=jax.ShapeDtypeStruct((UP_LEN,), jnp.float32),
        mesh=_mesh,
        compiler_params=_compiler_params(),
        scratch_types=[
            pltpu.VMEM((T,), jnp.float32),
            pltpu.VMEM((T,), jnp.float32),
            pltpu.VMEM((T,), jnp.float32),
            pltpu.VMEM((T,), jnp.float32),
            pltpu.VMEM((T,), jnp.int32),
            pltpu.VMEM((CHUNK,), jnp.float32),
        ],
    )


def kernel(f0):
    out = _make_call()(f0.reshape(T))
    return out.reshape(1, 1, UP_LEN)


# speedup vs baseline: 1.1149x; 1.1149x over previous
import dataclasses

import jax
import jax.numpy as jnp
from jax import lax
from jax.experimental import pallas as pl
from jax.experimental.pallas import tpu as pltpu
from jax.experimental.pallas import tpu_sc as plsc

T = 4096
UP_LEN = T * 256
M = UP_LEN - 1
NC = 2
NS = 16
L = 16
NW = NC * NS
CHUNK = UP_LEN // NW
NVEC = CHUNK // L
NBLK = T // L

_mesh = plsc.VectorSubcoreMesh(core_axis_name="c", subcore_axis_name="s")


KCH = 8
CVEC = NVEC // KCH


def _sc_kernel(f0_hbm, out_hbm, f0_v, xv, yv, dv, rank_v, out_v,
               x1t, hinvt, b0t, cct, eet):
    wid = lax.axis_index("s") * NC + lax.axis_index("c")
    pltpu.sync_copy(f0_hbm, f0_v)
    iota = lax.iota(jnp.int32, L)
    zero_i = jnp.zeros((L,), jnp.int32)
    lane0 = iota == 0

    def p1(c, carry):
        base = c * jnp.int32(L)
        vals = f0_v[pl.ds(base, L)]
        voiced = vals > 0.0
        v_i = voiced.astype(jnp.int32)
        rank_inc = jnp.cumsum(v_i) + carry
        rank_v[pl.ds(base, L)] = rank_inc - 1
        idx = jnp.maximum(rank_inc - 1, 0)
        pos_f = (iota + base).astype(jnp.float32)
        plsc.store_scatter(xv, [idx], pos_f, mask=voiced)
        plsc.store_scatter(yv, [idx], vals, mask=voiced)
        return jnp.max(rank_inc)

    n_v = lax.fori_loop(jnp.int32(0), jnp.int32(NBLK), p1, jnp.int32(0))

    def gat(ref, i_scalar):
        return plsc.load_gather(ref, [zero_i + i_scalar])

    base0 = wid * jnp.int32(CHUNK)
    nseg_sc = jnp.maximum(n_v - 2, 0)

    def q_of(i_s):
        p_s = i_s * jnp.int32(4095)
        hi_s = lax.shift_right_logical(p_s, jnp.int32(20))
        s_s = hi_s + (p_s & jnp.int32(M))
        return jnp.where(s_s >= M, hi_s + 1, hi_s)

    def seg_at(q_s):
        return jnp.clip(jnp.max(plsc.load_gather(rank_v, [zero_i + q_s])),
                        0, nseg_sc)

    slo_s = seg_at(q_of(base0))
    shi_s = seg_at(q_of(base0 + jnp.int32(CHUNK - 1)))
    b_lo = lax.shift_right_logical(slo_s, jnp.int32(4))
    b_hi = lax.shift_right_logical(shi_s + 1, jnp.int32(4)) + 1

    @plsc.parallel_loop(b_lo, b_hi, jnp.int32(1), unroll=1)
    def p2(c):
        base = c * jnp.int32(L)
        k = iota + base
        km1 = jnp.maximum(k - 1, 0)
        kp1 = jnp.minimum(k + 1, T - 1)
        xk = xv[pl.ds(base, L)]
        yk = yv[pl.ds(base, L)]
        xkm1 = plsc.load_gather(xv, [km1])
        xkp1 = plsc.load_gather(xv, [kp1])
        ykm1 = plsc.load_gather(yv, [km1])
        ykp1 = plsc.load_gather(yv, [kp1])
        dxk = xkp1 - xk
        dxkm1 = xk - xkm1
        dk = (ykp1 - yk) / dxk
        dkm1 = (yk - ykm1) / dxkm1
        w1 = 2.0 * dxk + dxkm1
        w2 = dxk + 2.0 * dxkm1
        mono = (dkm1 * dk) > 0.0
        skm1 = jnp.where(mono, dkm1, 1.0)
        sk = jnp.where(mono, dk, 1.0)
        hmean = (w1 + w2) / (w1 / skm1 + w2 / sk)
        dv[pl.ds(base, L)] = jnp.where(mono, hmean, 0.0)

    x0e = gat(xv, 0)
    x1e = gat(xv, 1)
    x2e = gat(xv, 2)
    y0e = gat(yv, 0)
    y1e = gat(yv, 1)
    y2e = gat(yv, 2)
    h0 = x1e - x0e
    h1 = x2e - x1e
    delta0 = (y1e - y0e) / h0
    delta1 = (y2e - y1e) / h1
    d0 = ((2.0 * h0 + h1) * delta0 - h0 * delta1) / (h0 + h1)
    mask0 = d0 * delta0 <= 0.0
    d0 = jnp.where(mask0, 0.0, d0)
    mask0b = (delta0 * delta1 < 0.0) & (~mask0)
    d0 = jnp.where(mask0b & (jnp.abs(d0) > 3.0 * jnp.abs(delta0)), 3.0 * delta0, d0)
    plsc.store_scatter(dv, [zero_i], d0, mask=lane0)

    m_i = jnp.maximum(n_v - 2, 0)
    mm1_i = jnp.maximum(n_v - 3, 0)
    xa = gat(xv, mm1_i)
    xb = gat(xv, jnp.minimum(mm1_i + 1, T - 1))
    xc = gat(xv, m_i)
    xd = gat(xv, jnp.minimum(m_i + 1, T - 1))
    ya = gat(yv, mm1_i)
    yb = gat(yv, jnp.minimum(mm1_i + 1, T - 1))
    yc = gat(yv, m_i)
    yd = gat(yv, jnp.minimum(m_i + 1, T - 1))
    hm2 = xb - xa
    hm1 = xd - xc
    dlast = (yd - yc) / hm1
    dlast2 = (yb - ya) / hm2
    dn = ((2.0 * hm1 + hm2) * dlast - hm1 * dlast2) / (hm1 + hm2)
    maskn = dn * dlast <= 0.0
    dn = jnp.where(maskn, 0.0, dn)
    masknb = (dlast * dlast2 < 0.0) & (~maskn)
    dn = jnp.where(masknb & (jnp.abs(dn) > 3.0 * jnp.abs(dlast)), 3.0 * dlast, dn)
    plsc.store_scatter(dv, [zero_i + jnp.maximum(n_v - 1, 0)], dn, mask=lane0)

    @pl.when(n_v == 2)
    def _():
        plsc.store_scatter(dv, [jnp.minimum(iota, 1)], delta0, mask=iota < 2)

    @plsc.parallel_loop(b_lo, b_hi, jnp.int32(1), unroll=1)
    def p3(c):
        base = c * jnp.int32(L)
        s1 = jnp.minimum(iota + base + 1, T - 1)
        x0s = xv[pl.ds(base, L)]
        y0s = yv[pl.ds(base, L)]
        d0s = dv[pl.ds(base, L)]
        x1s = plsc.load_gather(xv, [s1])
        y1s = plsc.load_gather(yv, [s1])
        d1s = plsc.load_gather(dv, [s1])
        hs = x1s - x0s
        dys = y1s - y0s
        b0s = hs * d0s
        b1s = hs * d1s
        x1t[pl.ds(base, L)] = x1s
        hinvt[pl.ds(base, L)] = 1.0 / hs
        b0t[pl.ds(base, L)] = b0s
        cct[pl.ds(base, L)] = 3.0 * dys - 2.0 * b0s - b1s
        eet[pl.ds(base, L)] = b0s + b1s - 2.0 * dys

    nseg_s = zero_i + nseg_sc
    nv_ok_b = (zero_i + (n_v >= 2).astype(jnp.int32)) == 1
    m_f = jnp.float32(M)

    def run_span(lo_vec, hi_vec):
        @plsc.parallel_loop(lo_vec, hi_vec, jnp.int32(1), unroll=4)
        def body(c):
            i = iota + (base0 + c * jnp.int32(L))
            p = i * 4095
            hi = lax.shift_right_logical(p, jnp.int32(20))
            lo = p & M
            sfold = hi + lo
            ge = sfold >= M
            r = jnp.where(ge, sfold - M, sfold)
            q = jnp.where(ge, hi + 1, hi)
            frac = r.astype(jnp.float32) / m_f
            seg = jnp.clip(plsc.load_gather(rank_v, [q]), 0, nseg_s)
            x0 = plsc.load_gather(xv, [seg])
            x1 = plsc.load_gather(x1t, [seg])
            y0 = plsc.load_gather(yv, [seg])
            b0 = plsc.load_gather(b0t, [seg])
            cc = plsc.load_gather(cct, [seg])
            ee = plsc.load_gather(eet, [seg])
            hin = plsc.load_gather(hinvt, [seg])
            qf = q.astype(jnp.float32)
            t = ((qf - x0) + frac) * hin
            up = ((ee * t + cc) * t + b0) * t + y0
            up = jnp.where(up < 0.0, 0.0, up)
            ux = qf + frac
            fi = ux.astype(jnp.int32)
            fr = ux - fi.astype(jnp.float32)
            gt = fr > 0.5
            tie_odd = (fr == 0.5) & ((fi & 1) == 1)
            back = (fi + (gt | tie_odd).astype(jnp.int32)).astype(jnp.float32)
            keep = ((back == x0) | (back == x1)) & nv_ok_b
            out_v[pl.ds(c * jnp.int32(L), L)] = jnp.where(keep, up, 0.0)

    run_span(jnp.int32(0), jnp.int32(NVEC))
    pltpu.sync_copy(out_v, out_hbm.at[pl.ds(base0, CHUNK)])


def _compiler_params():
    cp = pltpu.CompilerParams()
    if "needs_layout_passes" in pltpu.CompilerParams.__dataclass_fields__:
        cp = dataclasses.replace(cp, needs_layout_passes=False)
    return cp


def _make_call():
    return pl.kernel(
        _sc_kernel,
        out_type=jax.ShapeDtypeStruct((UP_LEN,), jnp.float32),
        mesh=_mesh,
        compiler_params=_compiler_params(),
        scratch_types=[
            pltpu.VMEM((T,), jnp.float32),
            pltpu.VMEM((T,), jnp.float32),
            pltpu.VMEM((T,), jnp.float32),
            pltpu.VMEM((T,), jnp.float32),
            pltpu.VMEM((T,), jnp.int32),
            pltpu.VMEM((CHUNK,), jnp.float32),
            pltpu.VMEM((T,), jnp.float32),
            pltpu.VMEM((T,), jnp.float32),
            pltpu.VMEM((T,), jnp.float32),
            pltpu.VMEM((T,), jnp.float32),
            pltpu.VMEM((T,), jnp.float32),
        ],
    )


def kernel(f0):
    out = _make_call()(f0.reshape(T))
    return out.reshape(1, 1, UP_LEN)


# speedup vs baseline: 1.1855x; 1.0634x over previous
import dataclasses

import jax
import jax.numpy as jnp
from jax import lax
from jax.experimental import pallas as pl
from jax.experimental.pallas import tpu as pltpu
from jax.experimental.pallas import tpu_sc as plsc

T = 4096
UP_LEN = T * 256
M = UP_LEN - 1
NC = 2
NS = 16
L = 16
NW = NC * NS
CHUNK = UP_LEN // NW
NVEC = CHUNK // L
NBLK = T // L

_mesh = plsc.VectorSubcoreMesh(core_axis_name="c", subcore_axis_name="s")


KCH = 8
CVEC = NVEC // KCH


def _sc_kernel(f0_hbm, out_hbm, f0_v, xv, yv, dv, rank_v, bs_v, offs_v, out_v,
               x1t, hinvt, b0t, cct, eet):
    wid = lax.axis_index("s") * NC + lax.axis_index("c")
    pltpu.sync_copy(f0_hbm, f0_v)
    iota = lax.iota(jnp.int32, L)
    zero_i = jnp.zeros((L,), jnp.int32)
    lane0 = iota == 0

    @plsc.parallel_loop(jnp.int32(0), jnp.int32(NBLK), jnp.int32(1), unroll=4)
    def p1a(c):
        vals = f0_v[pl.ds(c * jnp.int32(L), L)]
        cnt = plsc.all_reduce_population_count(vals > 0.0)
        plsc.store_scatter(bs_v, [zero_i + c], cnt, mask=lane0)

    def p1b(c, carry):
        base = c * jnp.int32(L)
        v = bs_v[pl.ds(base, L)]
        inc = jnp.cumsum(v) + carry
        offs_v[pl.ds(base, L)] = inc - v
        return jnp.max(inc)

    n_v = lax.fori_loop(jnp.int32(0), jnp.int32(NBLK // L), p1b, jnp.int32(0))

    @plsc.parallel_loop(jnp.int32(0), jnp.int32(NBLK), jnp.int32(1), unroll=4)
    def p1c(c):
        base = c * jnp.int32(L)
        vals = f0_v[pl.ds(base, L)]
        voiced = vals > 0.0
        off = plsc.load_gather(offs_v, [zero_i + c])
        rank_inc = jnp.cumsum(voiced.astype(jnp.int32)) + off
        rank_v[pl.ds(base, L)] = rank_inc - 1
        idx = jnp.maximum(rank_inc - 1, 0)
        pos_f = (iota + base).astype(jnp.float32)
        plsc.store_scatter(xv, [idx], pos_f, mask=voiced)
        plsc.store_scatter(yv, [idx], vals, mask=voiced)

    def gat(ref, i_scalar):
        return plsc.load_gather(ref, [zero_i + i_scalar])

    base0 = wid * jnp.int32(CHUNK)
    nseg_sc = jnp.maximum(n_v - 2, 0)

    def q_of(i_s):
        p_s = i_s * jnp.int32(4095)
        hi_s = lax.shift_right_logical(p_s, jnp.int32(20))
        s_s = hi_s + (p_s & jnp.int32(M))
        return jnp.where(s_s >= M, hi_s + 1, hi_s)

    def seg_at(q_s):
        return jnp.clip(jnp.max(plsc.load_gather(rank_v, [zero_i + q_s])),
                        0, nseg_sc)

    slo_s = seg_at(q_of(base0))
    shi_s = seg_at(q_of(base0 + jnp.int32(CHUNK - 1)))
    b_lo = lax.shift_right_logical(slo_s, jnp.int32(4))
    b_hi = lax.shift_right_logical(shi_s + 1, jnp.int32(4)) + 1

    @plsc.parallel_loop(b_lo, b_hi, jnp.int32(1), unroll=1)
    def p2(c):
        base = c * jnp.int32(L)
        k = iota + base
        km1 = jnp.maximum(k - 1, 0)
        kp1 = jnp.minimum(k + 1, T - 1)
        xk = xv[pl.ds(base, L)]
        yk = yv[pl.ds(base, L)]
        xkm1 = plsc.load_gather(xv, [km1])
        xkp1 = plsc.load_gather(xv, [kp1])
        ykm1 = plsc.load_gather(yv, [km1])
        ykp1 = plsc.load_gather(yv, [kp1])
        dxk = xkp1 - xk
        dxkm1 = xk - xkm1
        dk = (ykp1 - yk) / dxk
        dkm1 = (yk - ykm1) / dxkm1
        w1 = 2.0 * dxk + dxkm1
        w2 = dxk + 2.0 * dxkm1
        mono = (dkm1 * dk) > 0.0
        skm1 = jnp.where(mono, dkm1, 1.0)
        sk = jnp.where(mono, dk, 1.0)
        hmean = (w1 + w2) / (w1 / skm1 + w2 / sk)
        dv[pl.ds(base, L)] = jnp.where(mono, hmean, 0.0)

    x0e = gat(xv, 0)
    x1e = gat(xv, 1)
    x2e = gat(xv, 2)
    y0e = gat(yv, 0)
    y1e = gat(yv, 1)
    y2e = gat(yv, 2)
    h0 = x1e - x0e
    h1 = x2e - x1e
    delta0 = (y1e - y0e) / h0
    delta1 = (y2e - y1e) / h1
    d0 = ((2.0 * h0 + h1) * delta0 - h0 * delta1) / (h0 + h1)
    mask0 = d0 * delta0 <= 0.0
    d0 = jnp.where(mask0, 0.0, d0)
    mask0b = (delta0 * delta1 < 0.0) & (~mask0)
    d0 = jnp.where(mask0b & (jnp.abs(d0) > 3.0 * jnp.abs(delta0)), 3.0 * delta0, d0)
    plsc.store_scatter(dv, [zero_i], d0, mask=lane0)

    m_i = jnp.maximum(n_v - 2, 0)
    mm1_i = jnp.maximum(n_v - 3, 0)
    xa = gat(xv, mm1_i)
    xb = gat(xv, jnp.minimum(mm1_i + 1, T - 1))
    xc = gat(xv, m_i)
    xd = gat(xv, jnp.minimum(m_i + 1, T - 1))
    ya = gat(yv, mm1_i)
    yb = gat(yv, jnp.minimum(mm1_i + 1, T - 1))
    yc = gat(yv, m_i)
    yd = gat(yv, jnp.minimum(m_i + 1, T - 1))
    hm2 = xb - xa
    hm1 = xd - xc
    dlast = (yd - yc) / hm1
    dlast2 = (yb - ya) / hm2
    dn = ((2.0 * hm1 + hm2) * dlast - hm1 * dlast2) / (hm1 + hm2)
    maskn = dn * dlast <= 0.0
    dn = jnp.where(maskn, 0.0, dn)
    masknb = (dlast * dlast2 < 0.0) & (~maskn)
    dn = jnp.where(masknb & (jnp.abs(dn) > 3.0 * jnp.abs(dlast)), 3.0 * dlast, dn)
    plsc.store_scatter(dv, [zero_i + jnp.maximum(n_v - 1, 0)], dn, mask=lane0)

    @pl.when(n_v == 2)
    def _():
        plsc.store_scatter(dv, [jnp.minimum(iota, 1)], delta0, mask=iota < 2)

    @plsc.parallel_loop(b_lo, b_hi, jnp.int32(1), unroll=1)
    def p3(c):
        base = c * jnp.int32(L)
        s1 = jnp.minimum(iota + base + 1, T - 1)
        x0s = xv[pl.ds(base, L)]
        y0s = yv[pl.ds(base, L)]
        d0s = dv[pl.ds(base, L)]
        x1s = plsc.load_gather(xv, [s1])
        y1s = plsc.load_gather(yv, [s1])
        d1s = plsc.load_gather(dv, [s1])
        hs = x1s - x0s
        dys = y1s - y0s
        b0s = hs * d0s
        b1s = hs * d1s
        x1t[pl.ds(base, L)] = x1s
        hinvt[pl.ds(base, L)] = 1.0 / hs
        b0t[pl.ds(base, L)] = b0s
        cct[pl.ds(base, L)] = 3.0 * dys - 2.0 * b0s - b1s
        eet[pl.ds(base, L)] = b0s + b1s - 2.0 * dys

    nseg_s = zero_i + nseg_sc
    nv_ok_b = (zero_i + (n_v >= 2).astype(jnp.int32)) == 1
    m_f = jnp.float32(M)

    def run_span(lo_vec, hi_vec):
        @plsc.parallel_loop(lo_vec, hi_vec, jnp.int32(1), unroll=4)
        def body(c):
            i = iota + (base0 + c * jnp.int32(L))
            p = i * 4095
            hi = lax.shift_right_logical(p, jnp.int32(20))
            lo = p & M
            sfold = hi + lo
            ge = sfold >= M
            r = jnp.where(ge, sfold - M, sfold)
            q = jnp.where(ge, hi + 1, hi)
            frac = r.astype(jnp.float32) / m_f
            seg = jnp.clip(plsc.load_gather(rank_v, [q]), 0, nseg_s)
            x0 = plsc.load_gather(xv, [seg])
            x1 = plsc.load_gather(x1t, [seg])
            y0 = plsc.load_gather(yv, [seg])
            b0 = plsc.load_gather(b0t, [seg])
            cc = plsc.load_gather(cct, [seg])
            ee = plsc.load_gather(eet, [seg])
            hin = plsc.load_gather(hinvt, [seg])
            qf = q.astype(jnp.float32)
            t = ((qf - x0) + frac) * hin
            up = ((ee * t + cc) * t + b0) * t + y0
            up = jnp.where(up < 0.0, 0.0, up)
            ux = qf + frac
            fi = ux.astype(jnp.int32)
            fr = ux - fi.astype(jnp.float32)
            gt = fr > 0.5
            tie_odd = (fr == 0.5) & ((fi & 1) == 1)
            back = (fi + (gt | tie_odd).astype(jnp.int32)).astype(jnp.float32)
            keep = ((back == x0) | (back == x1)) & nv_ok_b
            out_v[pl.ds(c * jnp.int32(L), L)] = jnp.where(keep, up, 0.0)

    run_span(jnp.int32(0), jnp.int32(NVEC))
    pltpu.sync_copy(out_v, out_hbm.at[pl.ds(base0, CHUNK)])


def _compiler_params():
    cp = pltpu.CompilerParams()
    if "needs_layout_passes" in pltpu.CompilerParams.__dataclass_fields__:
        cp = dataclasses.replace(cp, needs_layout_passes=False)
    return cp


def _make_call():
    return pl.kernel(
        _sc_kernel,
        out_type=jax.ShapeDtypeStruct((UP_LEN,), jnp.float32),
        mesh=_mesh,
        compiler_params=_compiler_params(),
        scratch_types=[
            pltpu.VMEM((T,), jnp.float32),
            pltpu.VMEM((T,), jnp.float32),
            pltpu.VMEM((T,), jnp.float32),
            pltpu.VMEM((T,), jnp.float32),
            pltpu.VMEM((T,), jnp.int32),
            pltpu.VMEM((NBLK,), jnp.int32),
            pltpu.VMEM((NBLK,), jnp.int32),
            pltpu.VMEM((CHUNK,), jnp.float32),
            pltpu.VMEM((T,), jnp.float32),
            pltpu.VMEM((T,), jnp.float32),
            pltpu.VMEM((T,), jnp.float32),
            pltpu.VMEM((T,), jnp.float32),
            pltpu.VMEM((T,), jnp.float32),
        ],
    )


def kernel(f0):
    out = _make_call()(f0.reshape(T))
    return out.reshape(1, 1, UP_LEN)


# speedup vs baseline: 1.2285x; 1.0362x over previous
import dataclasses

import jax
import jax.numpy as jnp
from jax import lax
from jax.experimental import pallas as pl
from jax.experimental.pallas import tpu as pltpu
from jax.experimental.pallas import tpu_sc as plsc

T = 4096
UP_LEN = T * 256
M = UP_LEN - 1
NC = 2
NS = 16
L = 16
NW = NC * NS
CHUNK = UP_LEN // NW
NVEC = CHUNK // L
NBLK = T // L

_mesh = plsc.VectorSubcoreMesh(core_axis_name="c", subcore_axis_name="s")


KCH = 8
CVEC = NVEC // KCH


def _sc_kernel(f0_hbm, out_hbm, f0_v, xv, yv, dv, rank_v, bs_v, offs_v, out_v,
               x1t, hinvt, b0t, cct, eet):
    wid = lax.axis_index("s") * NC + lax.axis_index("c")
    pltpu.sync_copy(f0_hbm, f0_v)
    iota = lax.iota(jnp.int32, L)
    zero_i = jnp.zeros((L,), jnp.int32)
    iota4095 = iota * jnp.int32(4095)
    lane0 = iota == 0

    @plsc.parallel_loop(jnp.int32(0), jnp.int32(NBLK), jnp.int32(1), unroll=4)
    def p1a(c):
        vals = f0_v[pl.ds(c * jnp.int32(L), L)]
        cnt = plsc.all_reduce_population_count(vals > 0.0)
        plsc.store_scatter(bs_v, [zero_i + c], cnt, mask=lane0)

    def p1b(c, carry):
        base = c * jnp.int32(L)
        v = bs_v[pl.ds(base, L)]
        inc = jnp.cumsum(v) + carry
        offs_v[pl.ds(base, L)] = inc - v
        return jnp.max(inc)

    n_v = lax.fori_loop(jnp.int32(0), jnp.int32(NBLK // L), p1b, jnp.int32(0))

    @plsc.parallel_loop(jnp.int32(0), jnp.int32(NBLK), jnp.int32(1), unroll=4)
    def p1c(c):
        base = c * jnp.int32(L)
        vals = f0_v[pl.ds(base, L)]
        voiced = vals > 0.0
        off = plsc.load_gather(offs_v, [zero_i + c])
        rank_inc = jnp.cumsum(voiced.astype(jnp.int32)) + off
        rank_v[pl.ds(base, L)] = rank_inc - 1
        idx = jnp.maximum(rank_inc - 1, 0)
        pos_f = (iota + base).astype(jnp.float32)
        plsc.store_scatter(xv, [idx], pos_f, mask=voiced)
        plsc.store_scatter(yv, [idx], vals, mask=voiced)

    def gat(ref, i_scalar):
        return plsc.load_gather(ref, [zero_i + i_scalar])

    base0 = wid * jnp.int32(CHUNK)
    nseg_sc = jnp.maximum(n_v - 2, 0)

    def q_of(i_s):
        p_s = i_s * jnp.int32(4095)
        hi_s = lax.shift_right_logical(p_s, jnp.int32(20))
        s_s = hi_s + (p_s & jnp.int32(M))
        return jnp.where(s_s >= M, hi_s + 1, hi_s)

    def seg_at(q_s):
        return jnp.clip(jnp.max(plsc.load_gather(rank_v, [zero_i + q_s])),
                        0, nseg_sc)

    slo_s = seg_at(q_of(base0))
    shi_s = seg_at(q_of(base0 + jnp.int32(CHUNK - 1)))
    b_lo = lax.shift_right_logical(slo_s, jnp.int32(4))
    b_hi = lax.shift_right_logical(shi_s + 1, jnp.int32(4)) + 1

    @plsc.parallel_loop(b_lo, b_hi, jnp.int32(1), unroll=1)
    def p2(c):
        base = c * jnp.int32(L)
        k = iota + base
        km1 = jnp.maximum(k - 1, 0)
        kp1 = jnp.minimum(k + 1, T - 1)
        xk = xv[pl.ds(base, L)]
        yk = yv[pl.ds(base, L)]
        xkm1 = plsc.load_gather(xv, [km1])
        xkp1 = plsc.load_gather(xv, [kp1])
        ykm1 = plsc.load_gather(yv, [km1])
        ykp1 = plsc.load_gather(yv, [kp1])
        dxk = xkp1 - xk
        dxkm1 = xk - xkm1
        dk = (ykp1 - yk) / dxk
        dkm1 = (yk - ykm1) / dxkm1
        w1 = 2.0 * dxk + dxkm1
        w2 = dxk + 2.0 * dxkm1
        mono = (dkm1 * dk) > 0.0
        skm1 = jnp.where(mono, dkm1, 1.0)
        sk = jnp.where(mono, dk, 1.0)
        hmean = (w1 + w2) / (w1 / skm1 + w2 / sk)
        dv[pl.ds(base, L)] = jnp.where(mono, hmean, 0.0)

    x0e = gat(xv, 0)
    x1e = gat(xv, 1)
    x2e = gat(xv, 2)
    y0e = gat(yv, 0)
    y1e = gat(yv, 1)
    y2e = gat(yv, 2)
    h0 = x1e - x0e
    h1 = x2e - x1e
    delta0 = (y1e - y0e) / h0
    delta1 = (y2e - y1e) / h1
    d0 = ((2.0 * h0 + h1) * delta0 - h0 * delta1) / (h0 + h1)
    mask0 = d0 * delta0 <= 0.0
    d0 = jnp.where(mask0, 0.0, d0)
    mask0b = (delta0 * delta1 < 0.0) & (~mask0)
    d0 = jnp.where(mask0b & (jnp.abs(d0) > 3.0 * jnp.abs(delta0)), 3.0 * delta0, d0)
    plsc.store_scatter(dv, [zero_i], d0, mask=lane0)

    m_i = jnp.maximum(n_v - 2, 0)
    mm1_i = jnp.maximum(n_v - 3, 0)
    xa = gat(xv, mm1_i)
    xb = gat(xv, jnp.minimum(mm1_i + 1, T - 1))
    xc = gat(xv, m_i)
    xd = gat(xv, jnp.minimum(m_i + 1, T - 1))
    ya = gat(yv, mm1_i)
    yb = gat(yv, jnp.minimum(mm1_i + 1, T - 1))
    yc = gat(yv, m_i)
    yd = gat(yv, jnp.minimum(m_i + 1, T - 1))
    hm2 = xb - xa
    hm1 = xd - xc
    dlast = (yd - yc) / hm1
    dlast2 = (yb - ya) / hm2
    dn = ((2.0 * hm1 + hm2) * dlast - hm1 * dlast2) / (hm1 + hm2)
    maskn = dn * dlast <= 0.0
    dn = jnp.where(maskn, 0.0, dn)
    masknb = (dlast * dlast2 < 0.0) & (~maskn)
    dn = jnp.where(masknb & (jnp.abs(dn) > 3.0 * jnp.abs(dlast)), 3.0 * dlast, dn)
    plsc.store_scatter(dv, [zero_i + jnp.maximum(n_v - 1, 0)], dn, mask=lane0)

    @pl.when(n_v == 2)
    def _():
        plsc.store_scatter(dv, [jnp.minimum(iota, 1)], delta0, mask=iota < 2)

    @plsc.parallel_loop(b_lo, b_hi, jnp.int32(1), unroll=1)
    def p3(c):
        base = c * jnp.int32(L)
        s1 = jnp.minimum(iota + base + 1, T - 1)
        x0s = xv[pl.ds(base, L)]
        y0s = yv[pl.ds(base, L)]
        d0s = dv[pl.ds(base, L)]
        x1s = plsc.load_gather(xv, [s1])
        y1s = plsc.load_gather(yv, [s1])
        d1s = plsc.load_gather(dv, [s1])
        hs = x1s - x0s
        dys = y1s - y0s
        b0s = hs * d0s
        b1s = hs * d1s
        x1t[pl.ds(base, L)] = x1s
        hinvt[pl.ds(base, L)] = 1.0 / hs
        b0t[pl.ds(base, L)] = b0s
        cct[pl.ds(base, L)] = 3.0 * dys - 2.0 * b0s - b1s
        eet[pl.ds(base, L)] = b0s + b1s - 2.0 * dys

    nseg_s = zero_i + nseg_sc
    nv_ok_b = (zero_i + (n_v >= 2).astype(jnp.int32)) == 1
    m_f = jnp.float32(M)

    def run_span(lo_vec, hi_vec):
        @plsc.parallel_loop(lo_vec, hi_vec, jnp.int32(1), unroll=4)
        def body(c):
            p = (zero_i + (base0 + c * jnp.int32(L)) * jnp.int32(4095)) + iota4095
            hi = lax.shift_right_logical(p, jnp.int32(20))
            lo = p & M
            sfold = hi + lo
            ge = sfold >= M
            r = jnp.where(ge, sfold - M, sfold)
            q = jnp.where(ge, hi + 1, hi)
            frac = r.astype(jnp.float32) / m_f
            seg = jnp.clip(plsc.load_gather(rank_v, [q]), 0, nseg_s)
            x0 = plsc.load_gather(xv, [seg])
            x1 = plsc.load_gather(x1t, [seg])
            y0 = plsc.load_gather(yv, [seg])
            b0 = plsc.load_gather(b0t, [seg])
            cc = plsc.load_gather(cct, [seg])
            ee = plsc.load_gather(eet, [seg])
            hin = plsc.load_gather(hinvt, [seg])
            qf = q.astype(jnp.float32)
            t = ((qf - x0) + frac) * hin
            up = ((ee * t + cc) * t + b0) * t + y0
            ux = qf + frac
            fi = ux.astype(jnp.int32)
            fr = ux - fi.astype(jnp.float32)
            gt = fr > 0.5
            tie_odd = (fr == 0.5) & ((fi & 1) == 1)
            back = (fi + (gt | tie_odd).astype(jnp.int32)).astype(jnp.float32)
            keep = ((back == x0) | (back == x1)) & nv_ok_b & (up > 0.0)
            out_v[pl.ds(c * jnp.int32(L), L)] = jnp.where(keep, up, 0.0)

    run_span(jnp.int32(0), jnp.int32(NVEC))
    pltpu.sync_copy(out_v, out_hbm.at[pl.ds(base0, CHUNK)])


def _compiler_params():
    cp = pltpu.CompilerParams()
    if "needs_layout_passes" in pltpu.CompilerParams.__dataclass_fields__:
        cp = dataclasses.replace(cp, needs_layout_passes=False)
    return cp


def _make_call():
    return pl.kernel(
        _sc_kernel,
        out_type=jax.ShapeDtypeStruct((UP_LEN,), jnp.float32),
        mesh=_mesh,
        compiler_params=_compiler_params(),
        scratch_types=[
            pltpu.VMEM((T,), jnp.float32),
            pltpu.VMEM((T,), jnp.float32),
            pltpu.VMEM((T,), jnp.float32),
            pltpu.VMEM((T,), jnp.float32),
            pltpu.VMEM((T,), jnp.int32),
            pltpu.VMEM((NBLK,), jnp.int32),
            pltpu.VMEM((NBLK,), jnp.int32),
            pltpu.VMEM((CHUNK,), jnp.float32),
            pltpu.VMEM((T,), jnp.float32),
            pltpu.VMEM((T,), jnp.float32),
            pltpu.VMEM((T,), jnp.float32),
            pltpu.VMEM((T,), jnp.float32),
            pltpu.VMEM((T,), jnp.float32),
        ],
    )


def kernel(f0):
    out = _make_call()(f0.reshape(T))
    return out.reshape(1, 1, UP_LEN)
